# Initial kernel scaffold; baseline (speedup 1.0000x reference)
import jax
import jax.numpy as jnp
from jax.experimental import pallas as pl


def kernel(inputs, table, W, b):
    raise NotImplementedError("write your pallas kernel here")


# speedup vs baseline: 2.9977x; 2.9977x over previous
import functools

import jax
import jax.numpy as jnp
from jax import lax
from jax.experimental import pallas as pl
from jax.experimental.pallas import tpu as pltpu
from jax.experimental.pallas import tpu_sc as plsc

NC = 2
NS = 16
NW = NC * NS
GRP = 128


def _build_sc_gather(tot, emb):
    per_w = tot // NW
    ng = per_w // GRP

    mesh = plsc.VectorSubcoreMesh(
        core_axis_name="c", subcore_axis_name="s",
        num_cores=NC, num_subcores=NS)

    @functools.partial(
        pl.kernel,
        out_type=jax.ShapeDtypeStruct((tot, emb), jnp.float32),
        mesh=mesh,
        scratch_types=[
            pltpu.VMEM((ng, GRP), jnp.int32),
            pltpu.VMEM((GRP, emb), jnp.float32),
            pltpu.SemaphoreType.DMA,
        ],
        compiler_params=pltpu.CompilerParams(use_tc_tiling_on_sc=False),
    )
    def sc_gather(table_hbm, idx_hbm, out_hbm, idx_v, rows_v, sem):
        wid = lax.axis_index("s") * NC + lax.axis_index("c")
        base = wid * per_w
        pltpu.sync_copy(idx_hbm.at[wid], idx_v)

        @pl.loop(0, ng)
        def _(g):
            cp = pltpu.async_copy(table_hbm.at[idx_v.at[g]], rows_v, sem)
            cp.wait()
            pltpu.sync_copy(rows_v, out_hbm.at[pl.ds(base + g * GRP, GRP)])

    return sc_gather


def _mm_body(x_ref, w_ref, b_ref, o_ref):
    o_ref[...] = jnp.dot(
        x_ref[...], w_ref[...], preferred_element_type=jnp.float32
    ) + b_ref[...]


def _tc_matmul(x, w, b):
    m, k = x.shape
    n = w.shape[1]
    bm = 2048
    return pl.pallas_call(
        _mm_body,
        grid=(m // bm,),
        in_specs=[
            pl.BlockSpec((bm, k), lambda i: (i, 0)),
            pl.BlockSpec((k, n), lambda i: (0, 0)),
            pl.BlockSpec((1, n), lambda i: (0, 0)),
        ],
        out_specs=pl.BlockSpec((bm, n), lambda i: (i, 0)),
        out_shape=jax.ShapeDtypeStruct((m, n), jnp.float32),
    )(x, w, b.reshape(1, n))


def kernel(inputs, table, W, b):
    batch, seq = inputs.shape
    vocab, emb = table.shape
    d_model = W.shape[1]
    tot = batch * seq

    idx3d = inputs.reshape(NW, tot // (NW * GRP), GRP)
    gathered = _build_sc_gather(tot, emb)(table, idx3d)
    pairs = gathered.reshape(tot // 2, 2 * emb)
    out = _tc_matmul(pairs, W, b)
    return out.reshape(batch, seq // 2, d_model)


# speedup vs baseline: 3.4073x; 1.1366x over previous
import functools

import jax
import jax.numpy as jnp
from jax import lax
from jax.experimental import pallas as pl
from jax.experimental.pallas import tpu as pltpu
from jax.experimental.pallas import tpu_sc as plsc

NC = 2
NS = 16
NW = NC * NS
GRP = 128


def _build_sc_gather(tot, emb):
    per_w = tot // NW
    ng = per_w // GRP

    mesh = plsc.VectorSubcoreMesh(
        core_axis_name="c", subcore_axis_name="s",
        num_cores=NC, num_subcores=NS)

    nbuf = 5
    assert ng > nbuf and (ng - nbuf) % nbuf == 0

    @functools.partial(
        pl.kernel,
        out_type=jax.ShapeDtypeStruct((tot, emb), jnp.float32),
        mesh=mesh,
        scratch_types=[
            pltpu.VMEM((ng, GRP), jnp.int32),
            [pltpu.VMEM((GRP, emb), jnp.float32) for _ in range(nbuf)],
            [pltpu.SemaphoreType.DMA for _ in range(nbuf)],
            [pltpu.SemaphoreType.DMA for _ in range(nbuf)],
        ],
        compiler_params=pltpu.CompilerParams(use_tc_tiling_on_sc=False),
    )
    def sc_gather(table_hbm, idx_hbm, out_hbm, idx_v, rows, gsems, wsems):
        wid = lax.axis_index("s") * NC + lax.axis_index("c")
        base = wid * per_w
        pltpu.sync_copy(idx_hbm.at[wid], idx_v)

        def gather_start(grp, b):
            pltpu.async_copy(table_hbm.at[idx_v.at[grp]], rows[b], gsems[b])

        def gather_wait(grp, b):
            pltpu.make_async_copy(
                table_hbm.at[idx_v.at[grp]], rows[b], gsems[b]).wait()

        def wb_start(grp, b):
            pltpu.async_copy(
                rows[b], out_hbm.at[pl.ds(base + grp * GRP, GRP)], wsems[b])

        def wb_wait(grp, b):
            pltpu.make_async_copy(
                rows[b], out_hbm.at[pl.ds(base + grp * GRP, GRP)],
                wsems[b]).wait()

        for b in range(nbuf):
            gather_start(b, b)

        @pl.loop(0, ng - nbuf, step=nbuf)
        def _(g):
            for b in range(nbuf):
                grp = g + b
                gather_wait(grp, b)
                wb_start(grp, b)
                wb_wait(grp, b)
                gather_start(grp + nbuf, b)

        for b in range(nbuf):
            grp = ng - nbuf + b
            gather_wait(grp, b)
            wb_start(grp, b)
            wb_wait(grp, b)

    return sc_gather


def _mm_body(x_ref, w_ref, b_ref, o_ref):
    o_ref[...] = jnp.dot(
        x_ref[...], w_ref[...], preferred_element_type=jnp.float32
    ) + b_ref[...]


def _tc_matmul(x, w, b):
    m, k = x.shape
    n = w.shape[1]
    bm = 2048
    return pl.pallas_call(
        _mm_body,
        grid=(m // bm,),
        in_specs=[
            pl.BlockSpec((bm, k), lambda i: (i, 0)),
            pl.BlockSpec((k, n), lambda i: (0, 0)),
            pl.BlockSpec((1, n), lambda i: (0, 0)),
        ],
        out_specs=pl.BlockSpec((bm, n), lambda i: (i, 0)),
        out_shape=jax.ShapeDtypeStruct((m, n), jnp.float32),
    )(x, w, b.reshape(1, n))


def kernel(inputs, table, W, b):
    batch, seq = inputs.shape
    vocab, emb = table.shape
    d_model = W.shape[1]
    tot = batch * seq

    idx3d = inputs.reshape(NW, tot // (NW * GRP), GRP)
    gathered = _build_sc_gather(tot, emb)(table, idx3d)
    pairs = gathered.reshape(tot // 2, 2 * emb)
    out = _tc_matmul(pairs, W, b)
    return out.reshape(batch, seq // 2, d_model)


# speedup vs baseline: 3.9459x; 1.1581x over previous
import functools

import jax
import jax.numpy as jnp
from jax import lax
from jax.experimental import pallas as pl
from jax.experimental.pallas import tpu as pltpu
from jax.experimental.pallas import tpu_sc as plsc

NC = 2
NS = 16
NW = NC * NS
GRP = 128


def _build_sc_gather(tot, emb):
    per_w = tot // NW
    ng = per_w // GRP

    mesh = plsc.VectorSubcoreMesh(
        core_axis_name="c", subcore_axis_name="s",
        num_cores=NC, num_subcores=NS)

    nbuf = 5
    assert ng > nbuf and (ng - nbuf) % nbuf == 0

    @functools.partial(
        pl.kernel,
        out_type=jax.ShapeDtypeStruct((tot, emb), jnp.float32),
        mesh=mesh,
        scratch_types=[
            pltpu.VMEM((ng, GRP), jnp.int32),
            [pltpu.VMEM((GRP, emb), jnp.float32) for _ in range(nbuf)],
            [pltpu.SemaphoreType.DMA for _ in range(nbuf)],
            [pltpu.SemaphoreType.DMA for _ in range(nbuf)],
        ],
        compiler_params=pltpu.CompilerParams(use_tc_tiling_on_sc=False),
    )
    def sc_gather(table_hbm, idx_hbm, out_hbm, idx_v, rows, gsems, wsems):
        wid = lax.axis_index("s") * NC + lax.axis_index("c")
        base = wid * per_w
        pltpu.sync_copy(idx_hbm.at[wid], idx_v)

        def gather_start(grp, b):
            pltpu.async_copy(table_hbm.at[idx_v.at[grp]], rows[b], gsems[b])

        def gather_wait(grp, b):
            pltpu.make_async_copy(
                table_hbm.at[idx_v.at[grp]], rows[b], gsems[b]).wait()

        def wb_start(grp, b):
            pltpu.async_copy(
                rows[b], out_hbm.at[pl.ds(base + grp * GRP, GRP)], wsems[b])

        def wb_wait(grp, b):
            pltpu.make_async_copy(
                rows[b], out_hbm.at[pl.ds(base + grp * GRP, GRP)],
                wsems[b]).wait()

        for b in range(nbuf):
            gather_start(b, b)

        @pl.loop(0, ng - nbuf, step=nbuf)
        def _(g):
            for b in range(nbuf):
                grp = g + b
                gather_wait(grp, b)
                wb_start(grp, b)
                wb_wait(grp, b)
                gather_start(grp + nbuf, b)

        for b in range(nbuf):
            grp = ng - nbuf + b
            gather_wait(grp, b)
            wb_start(grp, b)
            wb_wait(grp, b)

    return sc_gather


def _mm_body(x_ref, w_ref, b_ref, o_ref):
    o_ref[...] = jnp.dot(
        x_ref[...], w_ref[...], preferred_element_type=jnp.float32
    ) + b_ref[...]


def _tc_matmul(x, w, b):
    m, k = x.shape
    n = w.shape[1]
    bm = 2048
    return pl.pallas_call(
        _mm_body,
        grid=(m // bm,),
        in_specs=[
            pl.BlockSpec((bm, k), lambda i: (i, 0)),
            pl.BlockSpec((k, n), lambda i: (0, 0)),
            pl.BlockSpec((1, n), lambda i: (0, 0)),
        ],
        out_specs=pl.BlockSpec((bm, n), lambda i: (i, 0)),
        out_shape=jax.ShapeDtypeStruct((m, n), jnp.float32),
    )(x, w, b.reshape(1, n))


def kernel(inputs, table, W, b):
    batch, seq = inputs.shape
    vocab, emb = table.shape
    d_model = W.shape[1]
    tot = batch * seq

    num_pairs = seq // 2
    idx_perm = inputs.reshape(batch, num_pairs, 2).transpose(1, 0, 2)
    idx3d = idx_perm.reshape(NW, tot // (NW * GRP), GRP)
    gathered = _build_sc_gather(tot, emb)(table, idx3d)
    pairs = gathered.reshape(tot // 2, 2 * emb)
    out = _tc_matmul(pairs, W, b)
    return out.reshape(num_pairs, batch, d_model).transpose(1, 0, 2)
